# baseline (speedup 1.0000x reference)
import functools

import jax
import jax.numpy as jnp
from jax import lax
from jax.experimental import pallas as pl
from jax.experimental.pallas import tpu as pltpu
from jax.experimental.pallas import tpu_sc as plsc

N_NODES, D_IN, K_NEIGH, H_MID, C_OUT = 10000, 128, 32, 128, 64
NW = 32
NSUB = 16
CH = 80
NPAD = 10240
CHUNKS_PER_W = NPAD // (CH * NW)
NSPLIT = 4
KG = K_NEIGH // NSPLIT
STG = NPAD // NSUB


def _gather_sum_body(table_hbm, adj_hbm, out_hbm, idx_v,
                     acc0, acc1, acc2, acc3, tbl_sh, sem):
    accs = (acc0, acc1, acc2, acc3)
    c = lax.axis_index("c")
    s = lax.axis_index("s")
    wid = s * 2 + c

    r = s * STG
    pltpu.sync_copy(table_hbm.at[pl.ds(r, STG)], tbl_sh.at[pl.ds(r, STG)])
    plsc.subcore_barrier()

    def chunk_body(i, carry):
        chunk = wid * CHUNKS_PER_W + i
        base = chunk * CH
        pltpu.sync_copy(adj_hbm.at[chunk], idx_v)
        lead = [
            pltpu.async_copy(tbl_sh.at[idx_v.at[g * KG]], accs[g], sem)
            for g in range(NSPLIT)
        ]
        for cp in lead:
            cp.wait()
        adds = [
            pltpu.async_copy(tbl_sh.at[idx_v.at[g * KG + j]], accs[g], sem,
                             add=True)
            for j in range(1, KG)
            for g in range(NSPLIT)
        ]
        for cp in adds:
            cp.wait()
        for g in range(NSPLIT):
            pltpu.sync_copy(accs[g], out_hbm.at[g, pl.ds(base, CH)])
        return carry

    lax.fori_loop(0, CHUNKS_PER_W, chunk_body, 0)


def _gather_sum(table, adj_chunks):
    mesh = plsc.VectorSubcoreMesh(core_axis_name="c", subcore_axis_name="s")
    f = pl.kernel(
        _gather_sum_body,
        out_type=jax.ShapeDtypeStruct((NSPLIT, NPAD, D_IN), jnp.float32),
        mesh=mesh,
        scratch_types=[
            pltpu.VMEM((K_NEIGH, CH), jnp.int32),
            pltpu.VMEM((CH, D_IN), jnp.float32),
            pltpu.VMEM((CH, D_IN), jnp.float32),
            pltpu.VMEM((CH, D_IN), jnp.float32),
            pltpu.VMEM((CH, D_IN), jnp.float32),
            pltpu.VMEM_SHARED((NPAD, D_IN), jnp.float32),
            pltpu.SemaphoreType.DMA,
        ],
    )
    return f(table, adj_chunks)


def _mma_body(x_ref, w_ref, o_ref):
    o_ref[...] = jnp.dot(x_ref[...], w_ref[...],
                         preferred_element_type=jnp.float32)


def _mma(xp, w, n_out, blk):
    din, dout = w.shape
    return pl.pallas_call(
        _mma_body,
        grid=(n_out // blk,),
        in_specs=[
            pl.BlockSpec((blk, din), lambda i: (i, 0)),
            pl.BlockSpec((din, dout), lambda i: (0, 0)),
        ],
        out_specs=pl.BlockSpec((blk, dout), lambda i: (i, 0)),
        out_shape=jax.ShapeDtypeStruct((n_out, dout), jnp.float32),
    )(xp, w)


def _mmc_body(b_ref, a_ref, wb_ref, o_ref, *, relu):
    agg = jnp.sum(a_ref[...], axis=0)
    acc = b_ref[...] + jnp.dot(agg, wb_ref[...],
                               preferred_element_type=jnp.float32)
    if relu:
        acc = jnp.maximum(acc, 0.0)
    o_ref[...] = acc


def _mmc(base, agg_parts, wb, relu, n_out, blk):
    din, dout = wb.shape
    return pl.pallas_call(
        functools.partial(_mmc_body, relu=relu),
        grid=(n_out // blk,),
        in_specs=[
            pl.BlockSpec((blk, dout), lambda i: (i, 0)),
            pl.BlockSpec((NSPLIT, blk, din), lambda i: (0, i, 0)),
            pl.BlockSpec((din, dout), lambda i: (0, 0)),
        ],
        out_specs=pl.BlockSpec((blk, dout), lambda i: (i, 0)),
        out_shape=jax.ShapeDtypeStruct((n_out, dout), jnp.float32),
    )(base, agg_parts, wb)


def kernel(x, adj_lists, W0, W1):
    adj_pad = jnp.pad(adj_lists, ((0, 0), (0, NPAD - N_NODES)))
    adj_chunks = adj_pad.reshape(K_NEIGH, NPAD // CH, CH).transpose(1, 0, 2)
    x_pad = jnp.pad(x, ((0, NPAD - N_NODES), (0, 0)))
    inv_k = jnp.float32(1.0 / K_NEIGH)
    w0a, w0b = W0[:D_IN], W0[D_IN:] * inv_k
    w1a, w1b = W1[:H_MID], W1[H_MID:] * inv_k

    xa = _mma(x_pad, w0a, n_out=NPAD, blk=512)
    agg0 = _gather_sum(x_pad, adj_chunks)
    h = _mmc(xa, agg0, w0b, relu=True, n_out=NPAD, blk=512)
    ha = _mma(h, w1a, n_out=NPAD, blk=512)
    agg1 = _gather_sum(h, adj_chunks)
    out = _mmc(ha, agg1, w1b, relu=False, n_out=N_NODES, blk=400)
    return out


# speedup vs baseline: 1.1020x; 1.1020x over previous
import functools

import jax
import jax.numpy as jnp
from jax import lax
from jax.experimental import pallas as pl
from jax.experimental.pallas import tpu as pltpu
from jax.experimental.pallas import tpu_sc as plsc

N_NODES, D_IN, K_NEIGH, H_MID, C_OUT = 10000, 128, 32, 128, 64
NW = 32
NSUB = 16
CH = 80
NPAD = 10240
CHUNKS_PER_W = NPAD // (CH * NW)
NSPLIT = 2
KG = K_NEIGH // NSPLIT
STG = NPAD // NSUB


def _gather_sum_body(table_hbm, adj_hbm, out_hbm, idx_v,
                     acc0, acc1, acc2, acc3, tbl_sh, gsem, osem, isem):
    acc_sets = ((acc0, acc1), (acc2, acc3))
    c = lax.axis_index("c")
    s = lax.axis_index("s")
    wid = s * 2 + c
    cbase = wid * CHUNKS_PER_W

    def fire_idx(i):
        return pltpu.async_copy(adj_hbm.at[cbase + i], idx_v.at[i % 2], isem)

    idx_cps = [None] * CHUNKS_PER_W
    idx_cps[0] = fire_idx(0)
    r = s * STG
    pltpu.sync_copy(table_hbm.at[pl.ds(r, STG)], tbl_sh.at[pl.ds(r, STG)])
    idx_cps[0].wait()
    if CHUNKS_PER_W > 1:
        idx_cps[1] = fire_idx(1)
    plsc.subcore_barrier()

    def fire_leads(i):
        accs = acc_sets[i % 2]
        return [
            pltpu.async_copy(tbl_sh.at[idx_v.at[i % 2, g * KG]], accs[g],
                             gsem)
            for g in range(NSPLIT)
        ]

    def fire_adds(i):
        accs = acc_sets[i % 2]
        return [
            pltpu.async_copy(tbl_sh.at[idx_v.at[i % 2, g * KG + j]], accs[g],
                             gsem, add=True)
            for j in range(1, KG)
            for g in range(NSPLIT)
        ]

    def fire_out(i):
        accs = acc_sets[i % 2]
        base = (cbase + i) * CH
        return [
            pltpu.async_copy(accs[g], out_hbm.at[g, pl.ds(base, CH)], osem)
            for g in range(NSPLIT)
        ]

    leads = fire_leads(0)
    outs = [None] * CHUNKS_PER_W
    for i in range(CHUNKS_PER_W):
        for cp in leads:
            cp.wait()
        adds = fire_adds(i)
        if i + 1 < CHUNKS_PER_W:
            idx_cps[i + 1].wait()
            if i >= 1:
                for cp in outs[i - 1]:
                    cp.wait()
            leads = fire_leads(i + 1)
        for cp in adds:
            cp.wait()
        if i + 2 < CHUNKS_PER_W:
            idx_cps[i + 2] = fire_idx(i + 2)
        outs[i] = fire_out(i)
    for i in (CHUNKS_PER_W - 2, CHUNKS_PER_W - 1):
        for cp in outs[i]:
            cp.wait()


def _gather_sum(table, adj_chunks):
    mesh = plsc.VectorSubcoreMesh(core_axis_name="c", subcore_axis_name="s")
    f = pl.kernel(
        _gather_sum_body,
        out_type=jax.ShapeDtypeStruct((NSPLIT, NPAD, D_IN), jnp.float32),
        mesh=mesh,
        scratch_types=(
            [pltpu.VMEM((2, K_NEIGH, CH), jnp.int32)]
            + [pltpu.VMEM((CH, D_IN), jnp.float32) for _ in range(4)]
            + [pltpu.VMEM_SHARED((NPAD, D_IN), jnp.float32),
               pltpu.SemaphoreType.DMA, pltpu.SemaphoreType.DMA,
               pltpu.SemaphoreType.DMA]
        ),
    )
    return f(table, adj_chunks)


def _mma_body(x_ref, w_ref, o_ref):
    o_ref[...] = jnp.dot(x_ref[...], w_ref[...],
                         preferred_element_type=jnp.float32)


def _mma(xp, w, n_out, blk):
    din, dout = w.shape
    return pl.pallas_call(
        _mma_body,
        grid=(n_out // blk,),
        in_specs=[
            pl.BlockSpec((blk, din), lambda i: (i, 0)),
            pl.BlockSpec((din, dout), lambda i: (0, 0)),
        ],
        out_specs=pl.BlockSpec((blk, dout), lambda i: (i, 0)),
        out_shape=jax.ShapeDtypeStruct((n_out, dout), jnp.float32),
    )(xp, w)


def _mmc_body(b_ref, a_ref, wb_ref, o_ref, *, relu):
    agg = jnp.sum(a_ref[...], axis=0)
    acc = b_ref[...] + jnp.dot(agg, wb_ref[...],
                               preferred_element_type=jnp.float32)
    if relu:
        acc = jnp.maximum(acc, 0.0)
    o_ref[...] = acc


def _mmc(base, agg_parts, wb, relu, n_out, blk):
    din, dout = wb.shape
    return pl.pallas_call(
        functools.partial(_mmc_body, relu=relu),
        grid=(n_out // blk,),
        in_specs=[
            pl.BlockSpec((blk, dout), lambda i: (i, 0)),
            pl.BlockSpec((NSPLIT, blk, din), lambda i: (0, i, 0)),
            pl.BlockSpec((din, dout), lambda i: (0, 0)),
        ],
        out_specs=pl.BlockSpec((blk, dout), lambda i: (i, 0)),
        out_shape=jax.ShapeDtypeStruct((n_out, dout), jnp.float32),
    )(base, agg_parts, wb)


def kernel(x, adj_lists, W0, W1):
    adj_pad = jnp.pad(adj_lists, ((0, 0), (0, NPAD - N_NODES)))
    adj_chunks = adj_pad.reshape(K_NEIGH, NPAD // CH, CH).transpose(1, 0, 2)
    x_pad = jnp.pad(x, ((0, NPAD - N_NODES), (0, 0)))
    inv_k = jnp.float32(1.0 / K_NEIGH)
    w0a, w0b = W0[:D_IN], W0[D_IN:] * inv_k
    w1a, w1b = W1[:H_MID], W1[H_MID:] * inv_k

    xa = _mma(x_pad, w0a, n_out=NPAD, blk=512)
    agg0 = _gather_sum(x_pad, adj_chunks)
    h = _mmc(xa, agg0, w0b, relu=True, n_out=NPAD, blk=512)
    ha = _mma(h, w1a, n_out=NPAD, blk=512)
    agg1 = _gather_sum(h, adj_chunks)
    out = _mmc(ha, agg1, w1b, relu=False, n_out=N_NODES, blk=400)
    return out
